# baseline (speedup 1.0000x reference)
import functools

import jax
import jax.numpy as jnp
from jax import lax
from jax.experimental import pallas as pl
from jax.experimental.pallas import tpu as pltpu
from jax.experimental.pallas import tpu_sc as plsc

_T, _D, _E, _K, _FF = 2048, 768, 8, 2, 2048
_TM = 256
_PP = _T * _K + _E * _TM
_NT = _PP // _TM
_NJ = 4
_FJ = _FF // _NJ
_NEG = -1e30


def _router_body(x_ref, wr_ref, br_ref, out_ref):
    l = jnp.dot(x_ref[...], wr_ref[...],
                preferred_element_type=jnp.float32) + br_ref[...]
    m = jnp.max(l, axis=1, keepdims=True)
    p = jnp.exp(l - m)
    lanes = lax.broadcasted_iota(jnp.int32, p.shape, 1)
    a1 = jnp.argmax(p, axis=1)[:, None]
    p1 = jnp.max(p, axis=1, keepdims=True)
    pm = jnp.where(lanes == a1, -1.0, p)
    a2 = jnp.argmax(pm, axis=1)[:, None]
    p2 = jnp.max(pm, axis=1, keepdims=True)
    s = p1 + p2 + 1e-20
    out_ref[...] = jnp.where(
        lanes == 0, a1.astype(jnp.float32),
        jnp.where(lanes == 1, a2.astype(jnp.float32),
                  jnp.where(lanes == 2, p1 / s,
                            jnp.where(lanes == 3, p2 / s, 0.0))))


def _router(flat, wr_pad, br_pad):
    return pl.pallas_call(
        _router_body,
        out_shape=jax.ShapeDtypeStruct((_T, 128), jnp.float32),
    )(flat, wr_pad, br_pad)


def _ffn_body(off_ref, nt_ref, tok_ref, tbl_ref, wg_ref, wu_ref, wd_ref,
              w_ref, out_ref, xs_scr):
    j = pl.program_id(0)
    e = pl.program_id(1)
    base = pl.multiple_of(off_ref[e], _TM)

    def tile_body(t, carry):
        rows = pl.ds(base + t * _TM, _TM)

        @pl.when(j == 0)
        def _dispatch():
            idx = tok_ref[rows, :]
            oh = (idx == lax.broadcasted_iota(jnp.int32, (_TM, _T), 1))
            xb = jnp.dot(oh.astype(jnp.float32), tbl_ref[...],
                         preferred_element_type=jnp.float32)
            xs_scr[rows, :] = xb.astype(jnp.bfloat16)

        xb = xs_scr[rows, :].astype(jnp.float32)
        g = jnp.dot(xb, wg_ref[0], preferred_element_type=jnp.float32)
        u = jnp.dot(xb, wu_ref[0], preferred_element_type=jnp.float32)
        h = g * jax.nn.sigmoid(g) * u * w_ref[rows, :]
        part = jnp.dot(h, wd_ref[0], preferred_element_type=jnp.float32)

        @pl.when(j == 0)
        def _init():
            out_ref[rows, :] = part

        @pl.when(j > 0)
        def _acc():
            out_ref[rows, :] = out_ref[rows, :] + part

        return carry

    lax.fori_loop(0, nt_ref[e], tile_body, 0)


def _ffn_grouped(off_pad, ntiles, tok_col, flat, wg, wu, wd, w_col):
    grid_spec = pltpu.PrefetchScalarGridSpec(
        num_scalar_prefetch=2,
        grid=(_NJ, _E),
        in_specs=[
            pl.BlockSpec((_PP, 1), lambda j, e, off, nt: (0, 0)),
            pl.BlockSpec((_T, _D), lambda j, e, off, nt: (0, 0)),
            pl.BlockSpec((1, _D, _FJ), lambda j, e, off, nt: (e, 0, j)),
            pl.BlockSpec((1, _D, _FJ), lambda j, e, off, nt: (e, 0, j)),
            pl.BlockSpec((1, _FJ, _D), lambda j, e, off, nt: (e, j, 0)),
            pl.BlockSpec((_PP, 1), lambda j, e, off, nt: (0, 0)),
        ],
        out_specs=pl.BlockSpec((_PP, _D), lambda j, e, off, nt: (0, 0)),
        scratch_shapes=[
            pltpu.VMEM((_PP, _D), jnp.bfloat16),
        ],
    )
    return pl.pallas_call(
        _ffn_body, grid_spec=grid_spec,
        out_shape=jax.ShapeDtypeStruct((_PP, _D), jnp.float32),
    )(off_pad, ntiles, tok_col, flat, wg, wu, wd, w_col)


def _sc_gather(table, idx):
    rows, dd = table.shape
    (batch,) = idx.shape
    info = plsc.get_sparse_core_info()
    nw = info.num_cores * info.num_subcores
    assert batch % (8 * nw) == 0 and dd % info.num_lanes == 0
    bw = batch // nw
    mesh = plsc.VectorSubcoreMesh(core_axis_name="c", subcore_axis_name="s")

    @functools.partial(
        pl.kernel, mesh=mesh,
        out_type=jax.ShapeDtypeStruct((batch, dd), table.dtype),
        scratch_types=[
            pltpu.VMEM((bw,), jnp.int32),
            pltpu.VMEM((bw, dd), table.dtype),
            pltpu.SemaphoreType.DMA,
        ],
    )
    def k(table_hbm, idx_hbm, out_hbm, idx_v, rows_v, sem):
        wid = lax.axis_index("s") * info.num_cores + lax.axis_index("c")
        base = wid * bw
        pltpu.sync_copy(idx_hbm.at[pl.ds(base, bw)], idx_v)
        pltpu.async_copy(table_hbm.at[idx_v], rows_v, sem).wait()
        pltpu.sync_copy(rows_v, out_hbm.at[pl.ds(base, bw)])

    return k(table, idx)


def _shared_body(x_ref, wg_ref, wu_ref, wd_ref, yp0_ref, yp1_ref, out_ref):
    xb = x_ref[...]
    g = jnp.dot(xb, wg_ref[...], preferred_element_type=jnp.float32)
    u = jnp.dot(xb, wu_ref[...], preferred_element_type=jnp.float32)
    h = g * jax.nn.sigmoid(g) * u
    y = jnp.dot(h, wd_ref[...], preferred_element_type=jnp.float32)
    out_ref[...] = y + yp0_ref[...] + yp1_ref[...]


def _shared(flat, wsg, wsu, wsd, yp):
    bt = 256
    nb = _T // bt
    return pl.pallas_call(
        _shared_body,
        grid=(nb,),
        in_specs=[
            pl.BlockSpec((bt, _D), lambda i: (i, 0)),
            pl.BlockSpec((_D, _FF), lambda i: (0, 0)),
            pl.BlockSpec((_D, _FF), lambda i: (0, 0)),
            pl.BlockSpec((_FF, _D), lambda i: (0, 0)),
            pl.BlockSpec((bt, _D), lambda i: (i, 0)),
            pl.BlockSpec((bt, _D), lambda i, _nb=nb: (i + _nb, 0)),
        ],
        out_specs=pl.BlockSpec((bt, _D), lambda i: (i, 0)),
        out_shape=jax.ShapeDtypeStruct((_T, _D), jnp.float32),
    )(flat, wsg, wsu, wsd, yp, yp)


def _routing_metadata(r):
    ids = r[:, :_K].astype(jnp.int32)
    w = r[:, _K:2 * _K]
    e_flat = ids.reshape(-1)
    onehot = (e_flat[:, None] == jnp.arange(_E, dtype=jnp.int32)[None, :])
    oh32 = onehot.astype(jnp.int32)
    csum = jnp.cumsum(oh32, axis=0)
    rank = jnp.sum(csum * oh32, axis=1) - 1
    counts = csum[-1]
    pcounts = ((counts + _TM - 1) // _TM) * _TM
    off = jnp.concatenate(
        [jnp.zeros((1,), jnp.int32), jnp.cumsum(pcounts).astype(jnp.int32)])
    dest = jnp.sum(off[None, :_E] * oh32, axis=1) + rank
    tok = jnp.arange(_T * _K, dtype=jnp.int32) // _K
    w_bits = lax.bitcast_convert_type(w.reshape(-1), jnp.int32)
    packed = jnp.zeros((_PP, 2), jnp.int32).at[dest].set(
        jnp.stack([tok, w_bits], axis=-1))
    sorted_tok = packed[:, 0]
    w_sorted = lax.bitcast_convert_type(packed[:, 1], jnp.float32)
    ntiles = (counts + _TM - 1) // _TM
    dest_k = dest.reshape(_T, _K).T.reshape(-1)
    return dest_k, sorted_tok, w_sorted, off[:_E], ntiles.astype(jnp.int32)


def kernel(x, Wr, br, Wg, Wu, Wd, Wsg, Wsu, Wsd):
    b, s, d = x.shape
    flat = x.reshape(-1, d)
    wr_pad = jnp.zeros((_D, 128), jnp.float32).at[:, :_E].set(Wr)
    br_pad = jnp.full((1, 128), _NEG, jnp.float32).at[0, :_E].set(br)
    r = _router(flat, wr_pad, br_pad)
    dest_k, sorted_tok, w_sorted, off_pad, ntiles = _routing_metadata(r)

    ys = _ffn_grouped(off_pad, ntiles, sorted_tok.reshape(_PP, 1), flat,
                      Wg, Wu, Wd, w_sorted.reshape(_PP, 1))
    yp = _sc_gather(ys, dest_k)
    y = _shared(flat, Wsg, Wsu, Wsd, yp)
    return y.reshape(b, s, d)


# speedup vs baseline: 1.0268x; 1.0268x over previous
import functools

import jax
import jax.numpy as jnp
from jax import lax
from jax.experimental import pallas as pl
from jax.experimental.pallas import tpu as pltpu
from jax.experimental.pallas import tpu_sc as plsc

_T, _D, _E, _K, _FF = 2048, 768, 8, 2, 2048
_TM = 128
_PP = _T * _K + _E * _TM
_NT = _PP // _TM
_NJ = 2
_FJ = _FF // _NJ
_NEG = -1e30


def _router_body(x_ref, wr_ref, br_ref, out_ref):
    l = jnp.dot(x_ref[...], wr_ref[...],
                preferred_element_type=jnp.float32) + br_ref[...]
    m = jnp.max(l, axis=1, keepdims=True)
    p = jnp.exp(l - m)
    lanes = lax.broadcasted_iota(jnp.int32, p.shape, 1)
    a1 = jnp.argmax(p, axis=1)[:, None]
    p1 = jnp.max(p, axis=1, keepdims=True)
    pm = jnp.where(lanes == a1, -1.0, p)
    a2 = jnp.argmax(pm, axis=1)[:, None]
    p2 = jnp.max(pm, axis=1, keepdims=True)
    s = p1 + p2 + 1e-20
    out_ref[...] = jnp.where(
        lanes == 0, a1.astype(jnp.float32),
        jnp.where(lanes == 1, a2.astype(jnp.float32),
                  jnp.where(lanes == 2, p1 / s,
                            jnp.where(lanes == 3, p2 / s, 0.0))))


def _router(flat, wr_pad, br_pad):
    return pl.pallas_call(
        _router_body,
        out_shape=jax.ShapeDtypeStruct((_T, 128), jnp.float32),
    )(flat, wr_pad, br_pad)


def _ffn_body(off_ref, nt_ref, tok_ref, tbl_ref, wg_ref, wu_ref, wd_ref,
              w_ref, out_ref, xs_scr):
    j = pl.program_id(0)
    e = pl.program_id(1)
    base = pl.multiple_of(off_ref[e], _TM)

    def tile_body(t, carry):
        rows = pl.ds(base + t * _TM, _TM)

        @pl.when(j == 0)
        def _dispatch():
            idx = tok_ref[rows, :]
            oh = (idx == lax.broadcasted_iota(jnp.int32, (_TM, _T), 1))
            xb = jnp.dot(oh.astype(jnp.float32), tbl_ref[...],
                         preferred_element_type=jnp.float32)
            xs_scr[rows, :] = xb.astype(jnp.bfloat16)

        xb = xs_scr[rows, :].astype(jnp.float32)
        g = jnp.dot(xb, wg_ref[0], preferred_element_type=jnp.float32)
        u = jnp.dot(xb, wu_ref[0], preferred_element_type=jnp.float32)
        h = g * jax.nn.sigmoid(g) * u * w_ref[rows, :]
        part = jnp.dot(h, wd_ref[0], preferred_element_type=jnp.float32)

        @pl.when(j == 0)
        def _init():
            out_ref[rows, :] = part

        @pl.when(j > 0)
        def _acc():
            out_ref[rows, :] = out_ref[rows, :] + part

        return carry

    lax.fori_loop(0, nt_ref[e], tile_body, 0)


def _ffn_grouped(off_pad, ntiles, tok_col, flat, wg, wu, wd, w_col):
    grid_spec = pltpu.PrefetchScalarGridSpec(
        num_scalar_prefetch=2,
        grid=(_NJ, _E),
        in_specs=[
            pl.BlockSpec((_PP, 1), lambda j, e, off, nt: (0, 0)),
            pl.BlockSpec((_T, _D), lambda j, e, off, nt: (0, 0)),
            pl.BlockSpec((1, _D, _FJ), lambda j, e, off, nt: (e, 0, j)),
            pl.BlockSpec((1, _D, _FJ), lambda j, e, off, nt: (e, 0, j)),
            pl.BlockSpec((1, _FJ, _D), lambda j, e, off, nt: (e, j, 0)),
            pl.BlockSpec((_PP, 1), lambda j, e, off, nt: (0, 0)),
        ],
        out_specs=pl.BlockSpec((_PP, _D), lambda j, e, off, nt: (0, 0)),
        scratch_shapes=[
            pltpu.VMEM((_PP, _D), jnp.bfloat16),
        ],
    )
    return pl.pallas_call(
        _ffn_body, grid_spec=grid_spec,
        out_shape=jax.ShapeDtypeStruct((_PP, _D), jnp.float32),
    )(off_pad, ntiles, tok_col, flat, wg, wu, wd, w_col)


def _sc_gather(table, idx):
    rows, dd = table.shape
    (batch,) = idx.shape
    info = plsc.get_sparse_core_info()
    nw = info.num_cores * info.num_subcores
    assert batch % (8 * nw) == 0 and dd % info.num_lanes == 0
    bw = batch // nw
    mesh = plsc.VectorSubcoreMesh(core_axis_name="c", subcore_axis_name="s")

    @functools.partial(
        pl.kernel, mesh=mesh,
        out_type=jax.ShapeDtypeStruct((batch, dd), table.dtype),
        scratch_types=[
            pltpu.VMEM((bw,), jnp.int32),
            pltpu.VMEM((bw, dd), table.dtype),
            pltpu.SemaphoreType.DMA,
        ],
    )
    def k(table_hbm, idx_hbm, out_hbm, idx_v, rows_v, sem):
        wid = lax.axis_index("s") * info.num_cores + lax.axis_index("c")
        base = wid * bw
        pltpu.sync_copy(idx_hbm.at[pl.ds(base, bw)], idx_v)
        pltpu.async_copy(table_hbm.at[idx_v], rows_v, sem).wait()
        pltpu.sync_copy(rows_v, out_hbm.at[pl.ds(base, bw)])

    return k(table, idx)


def _shared_body(x_ref, wg_ref, wu_ref, wd_ref, yp0_ref, yp1_ref, out_ref):
    xb = x_ref[...]
    g = jnp.dot(xb, wg_ref[...], preferred_element_type=jnp.float32)
    u = jnp.dot(xb, wu_ref[...], preferred_element_type=jnp.float32)
    h = g * jax.nn.sigmoid(g) * u
    y = jnp.dot(h, wd_ref[...], preferred_element_type=jnp.float32)
    out_ref[...] = y + yp0_ref[...] + yp1_ref[...]


def _shared(flat, wsg, wsu, wsd, yp):
    bt = 256
    nb = _T // bt
    return pl.pallas_call(
        _shared_body,
        grid=(nb,),
        in_specs=[
            pl.BlockSpec((bt, _D), lambda i: (i, 0)),
            pl.BlockSpec((_D, _FF), lambda i: (0, 0)),
            pl.BlockSpec((_D, _FF), lambda i: (0, 0)),
            pl.BlockSpec((_FF, _D), lambda i: (0, 0)),
            pl.BlockSpec((bt, _D), lambda i: (i, 0)),
            pl.BlockSpec((bt, _D), lambda i, _nb=nb: (i + _nb, 0)),
        ],
        out_specs=pl.BlockSpec((bt, _D), lambda i: (i, 0)),
        out_shape=jax.ShapeDtypeStruct((_T, _D), jnp.float32),
    )(flat, wsg, wsu, wsd, yp, yp)


def _routing_metadata(r):
    ids = r[:, :_K].astype(jnp.int32)
    w = r[:, _K:2 * _K]
    e_flat = ids.reshape(-1)
    onehot = (e_flat[:, None] == jnp.arange(_E, dtype=jnp.int32)[None, :])
    oh32 = onehot.astype(jnp.int32)
    csum = jnp.cumsum(oh32, axis=0)
    rank = jnp.sum(csum * oh32, axis=1) - 1
    counts = csum[-1]
    pcounts = ((counts + _TM - 1) // _TM) * _TM
    off = jnp.concatenate(
        [jnp.zeros((1,), jnp.int32), jnp.cumsum(pcounts).astype(jnp.int32)])
    dest = jnp.sum(off[None, :_E] * oh32, axis=1) + rank
    tok = jnp.arange(_T * _K, dtype=jnp.int32) // _K
    w_bits = lax.bitcast_convert_type(w.reshape(-1), jnp.int32)
    packed = jnp.zeros((_PP, 2), jnp.int32).at[dest].set(
        jnp.stack([tok, w_bits], axis=-1))
    sorted_tok = packed[:, 0]
    w_sorted = lax.bitcast_convert_type(packed[:, 1], jnp.float32)
    ntiles = (counts + _TM - 1) // _TM
    dest_k = dest.reshape(_T, _K).T.reshape(-1)
    return dest_k, sorted_tok, w_sorted, off[:_E], ntiles.astype(jnp.int32)


def kernel(x, Wr, br, Wg, Wu, Wd, Wsg, Wsu, Wsd):
    b, s, d = x.shape
    flat = x.reshape(-1, d)
    wr_pad = jnp.zeros((_D, 128), jnp.float32).at[:, :_E].set(Wr)
    br_pad = jnp.full((1, 128), _NEG, jnp.float32).at[0, :_E].set(br)
    r = _router(flat, wr_pad, br_pad)
    dest_k, sorted_tok, w_sorted, off_pad, ntiles = _routing_metadata(r)

    ys = _ffn_grouped(off_pad, ntiles, sorted_tok.reshape(_PP, 1), flat,
                      Wg, Wu, Wd, w_sorted.reshape(_PP, 1))
    yp = _sc_gather(ys, dest_k)
    y = _shared(flat, Wsg, Wsu, Wsd, yp)
    return y.reshape(b, s, d)
